# Initial kernel scaffold; baseline (speedup 1.0000x reference)
import jax
import jax.numpy as jnp
from jax.experimental import pallas as pl


def kernel(feature_map, conv1_w, conv1_b, cls_w, cls_b, reg_w, reg_b):
    raise NotImplementedError("write your pallas kernel here")


# speedup vs baseline: 9.3981x; 9.3981x over previous
import functools

import jax
import jax.numpy as jnp
import numpy as np
from jax.experimental import pallas as pl
from jax.experimental.pallas import tpu as pltpu

_STRIDE = 16
_FH = _FW = 48
_C = 512
_HID = 2048
_IMG = float(_FH * _STRIDE)
_PRE = 2000
_POST = 300
_IOU = 0.7
_NA = _FH * _FW * 9


def _make_anchors(h, w):
    sizes = np.array([128.0, 256.0, 512.0])
    ratios = np.array([0.5, 1.0, 2.0])
    ws = (sizes[None, :] / np.sqrt(ratios)[:, None]).reshape(-1)
    hs = (sizes[None, :] * np.sqrt(ratios)[:, None]).reshape(-1)
    cx = (np.arange(w) + 0.5) * _STRIDE
    cy = (np.arange(h) + 0.5) * _STRIDE
    cxg, cyg = np.meshgrid(cx, cy)
    cxg = np.broadcast_to(cxg[..., None], (h, w, 9))
    cyg = np.broadcast_to(cyg[..., None], (h, w, 9))
    aw = np.broadcast_to(ws, (h, w, 9))
    ah = np.broadcast_to(hs, (h, w, 9))
    return np.stack([cxg, cyg, aw, ah], axis=-1).reshape(-1, 4).astype(np.float32)


_ANCHORS = _make_anchors(_FH, _FW)


def _conv_body(x_ref, w_ref, b_ref, h_ref):
    acc = jax.lax.dot_general(
        x_ref[...], w_ref[...], (((1,), (0,)), ((), ())),
        preferred_element_type=jnp.float32)
    h_ref[...] = jnp.maximum(acc + b_ref[...], 0.0)


def _heads_body(h_ref, wh_ref, bh_ref, l_ref):
    acc = jax.lax.dot_general(
        h_ref[...], wh_ref[...], (((1,), (0,)), ((), ())),
        preferred_element_type=jnp.float32)
    l_ref[...] = acc + bh_ref[...]


@functools.partial(jax.jit, static_argnames=())
def _conv_heads(x, w, b1, wh, bh):
    m_blk, n_blk = 256, 1024
    hidden = pl.pallas_call(
        _conv_body,
        grid=(_HID // n_blk, 2304 // m_blk),
        in_specs=[
            pl.BlockSpec((m_blk, 4608), lambda n, m: (m, 0)),
            pl.BlockSpec((4608, n_blk), lambda n, m: (0, n)),
            pl.BlockSpec((1, n_blk), lambda n, m: (0, n)),
        ],
        out_specs=pl.BlockSpec((m_blk, n_blk), lambda n, m: (m, n)),
        out_shape=jax.ShapeDtypeStruct((2304, _HID), jnp.float32),
        compiler_params=pltpu.CompilerParams(
            dimension_semantics=("arbitrary", "arbitrary")),
    )(x, w, b1)
    logits = pl.pallas_call(
        _heads_body,
        grid=(2304 // m_blk,),
        in_specs=[
            pl.BlockSpec((m_blk, _HID), lambda m: (m, 0)),
            pl.BlockSpec((_HID, 128), lambda m: (0, 0)),
            pl.BlockSpec((1, 128), lambda m: (0, 0)),
        ],
        out_specs=pl.BlockSpec((m_blk, 128), lambda m: (m, 0)),
        out_shape=jax.ShapeDtypeStruct((2304, 128), jnp.float32),
    )(hidden, wh, bh)
    return logits


_NPAD = 21504
_ROWS = _NPAD // 128


def _nms_body(cl_ref, o0_ref, o1_ref, o2_ref, o3_ref,
              acx_ref, acy_ref, aw_ref, ah_ref,
              out_ref, y1s, x1s, y2s, x2s, ars):
    shape = (_ROWS, 128)
    riota = jax.lax.broadcasted_iota(jnp.int32, shape, 0)
    liota = jax.lax.broadcasted_iota(jnp.int32, shape, 1)
    fiota = riota * 128 + liota

    s = jax.nn.sigmoid(cl_ref[...])
    o2 = jnp.clip(o2_ref[...], -4.0, 4.0)
    o3 = jnp.clip(o3_ref[...], -4.0, 4.0)
    aw = aw_ref[...]
    ah = ah_ref[...]
    cx = acx_ref[...] + aw * o0_ref[...]
    cy = acy_ref[...] + ah * o1_ref[...]
    pw = aw * jnp.exp(o2)
    ph = ah * jnp.exp(o3)
    x1 = jnp.clip(cx - pw / 2, 0.0, _IMG)
    y1 = jnp.clip(cy - ph / 2, 0.0, _IMG)
    x2 = jnp.clip(cx + pw / 2, 0.0, _IMG)
    y2 = jnp.clip(cy + ph / 2, 0.0, _IMG)
    y1s[...] = y1
    x1s[...] = x1
    y2s[...] = y2
    x2s[...] = x2
    ars[...] = (y2 - y1) * (x2 - x1)

    v = jax.lax.bitcast_convert_type(s, jnp.int32)
    v = jnp.where(fiota < _NA, v, jnp.int32(-1))

    def bis(_, lh):
        lo, hi = lh
        mid = lo + (hi - lo) // 2
        cnt = jnp.sum((v >= mid).astype(jnp.int32))
        big = cnt >= _PRE
        return (jnp.where(big, mid, lo), jnp.where(big, hi, mid))

    lo, hi = jax.lax.fori_loop(
        0, 32, bis, (jnp.int32(-2), jnp.int32(0x3F800001)))
    t = lo
    c_gt = jnp.sum((v > t).astype(jnp.int32))
    need = _PRE - c_gt

    def bis2(_, lh):
        lo2, hi2 = lh
        mid = lo2 + (hi2 - lo2) // 2
        cnt = jnp.sum(((v == t) & (fiota < mid)).astype(jnp.int32))
        enough = cnt >= need
        return (jnp.where(enough, lo2, mid), jnp.where(enough, mid, hi2))

    _, cut = jax.lax.fori_loop(
        0, 16, bis2, (jnp.int32(0), jnp.int32(_NPAD)))
    sel = (v > t) | ((v == t) & (fiota < cut))

    ninf = jnp.float32(-jnp.inf)
    ms0 = jnp.where(sel, s, ninf)
    out_ref[...] = jnp.zeros((304, 128), jnp.float32)
    l128 = jax.lax.broadcasted_iota(jnp.int32, (1, 128), 1)

    def step(i, ms):
        m = jnp.max(ms)
        valid = m > ninf
        flat = jnp.min(jnp.where(ms == m, fiota, jnp.int32(_NPAD)))
        r = flat // 128
        c = flat % 128

        def pick(ref):
            row = ref[pl.ds(r, 1), :]
            return jnp.sum(jnp.where(l128 == c, row, 0.0))

        by1 = pick(y1s)
        bx1 = pick(x1s)
        by2 = pick(y2s)
        bx2 = pick(x2s)
        ai = pick(ars)
        yy1 = jnp.maximum(y1s[...], by1)
        xx1 = jnp.maximum(x1s[...], bx1)
        yy2 = jnp.minimum(y2s[...], by2)
        xx2 = jnp.minimum(x2s[...], bx2)
        inter = jnp.maximum(yy2 - yy1, 0.0) * jnp.maximum(xx2 - xx1, 0.0)
        iou = inter / (ars[...] + ai - inter + 1e-9)
        ms = jnp.where((iou > _IOU) | (fiota == flat), ninf, ms)
        vf = jnp.where(valid, 1.0, 0.0).astype(jnp.float32)
        row = (jnp.where(l128 == 0, by1, 0.0) + jnp.where(l128 == 1, bx1, 0.0)
               + jnp.where(l128 == 2, by2, 0.0)
               + jnp.where(l128 == 3, bx2, 0.0)) * vf
        out_ref[pl.ds(i, 1), :] = row
        return ms

    jax.lax.fori_loop(0, _POST, step, ms0)


def _nms_call(cl, o0, o1, o2, o3, acx, acy, aw, ah):
    spec = pl.BlockSpec((_ROWS, 128), lambda: (0, 0))
    return pl.pallas_call(
        _nms_body,
        grid=(),
        in_specs=[spec] * 9,
        out_specs=pl.BlockSpec((304, 128), lambda: (0, 0)),
        out_shape=jax.ShapeDtypeStruct((304, 128), jnp.float32),
        scratch_shapes=[pltpu.VMEM((_ROWS, 128), jnp.float32)] * 5,
    )(cl, o0, o1, o2, o3, acx, acy, aw, ah)


def _pad_plane(x, fill):
    return jnp.concatenate(
        [x, jnp.full((_NPAD - _NA,), fill, jnp.float32)]).reshape(_ROWS, 128)


_APLANES = [
    np.concatenate([_ANCHORS[:, j],
                    np.ones(_NPAD - _NA, np.float32)]).reshape(_ROWS, 128)
    for j in range(4)
]


def kernel(feature_map, conv1_w, conv1_b, cls_w, cls_b, reg_w, reg_b):
    fmp = jnp.pad(feature_map[0], ((1, 1), (1, 1), (0, 0)))
    parts = [fmp[dy:dy + _FH, dx:dx + _FW, :].reshape(_FH * _FW, _C)
             for dy in range(3) for dx in range(3)]
    x = jnp.concatenate(parts, axis=1)
    w = conv1_w.reshape(9 * _C, _HID)
    wh = jnp.zeros((_HID, 128), jnp.float32)
    wh = wh.at[:, :9].set(cls_w.reshape(_HID, 9))
    wh = wh.at[:, 9:45].set(reg_w.reshape(_HID, 36))
    bh = jnp.zeros((1, 128), jnp.float32)
    bh = bh.at[0, :9].set(cls_b)
    bh = bh.at[0, 9:45].set(reg_b)

    logits = _conv_heads(x, w, conv1_b.reshape(1, _HID), wh, bh)

    cl = _pad_plane(logits[:, :9].reshape(-1), -1e30)
    op = logits[:, 9:45].reshape(-1, 4)
    o0 = _pad_plane(op[:, 0], 0.0)
    o1 = _pad_plane(op[:, 1], 0.0)
    o2 = _pad_plane(op[:, 2], 0.0)
    o3 = _pad_plane(op[:, 3], 0.0)
    a = [jnp.asarray(p) for p in _APLANES]

    out = _nms_call(cl, o0, o1, o2, o3, a[0], a[1], a[2], a[3])
    return out[:_POST, :4]
